# baseline (speedup 1.0000x reference)
import functools

import jax
import jax.numpy as jnp
from jax.experimental import pallas as pl
from jax.experimental.pallas import tpu as pltpu

_EPS = 1e-5
_IPB = 2
_IPB_TAIL = 4
_VMEM_LIMIT = 64 * 1024 * 1024
_CP = getattr(pltpu, "CompilerParams", None) or getattr(
    pltpu, "TPUCompilerParams")


def _cparams(sem):
    return _CP(dimension_semantics=sem, vmem_limit_bytes=_VMEM_LIMIT)


def _build_cols(xb, W, cols_ref):
    C, HW = xb.shape
    col = jax.lax.broadcasted_iota(jnp.int32, (1, HW), 1) % W
    has_l = col != 0
    has_r = col != (W - 1)
    zrow = jnp.zeros((C, W), jnp.bfloat16)
    z1 = jnp.zeros((C, 1), jnp.bfloat16)
    zb = jnp.zeros((C, HW), jnp.bfloat16)
    taps = (
        jnp.where(has_l, jnp.concatenate([z1, xb[:, :HW - 1]], axis=1), zb),
        xb,
        jnp.where(has_r, jnp.concatenate([xb[:, 1:], z1], axis=1), zb),
    )
    for kw in range(3):
        t = taps[kw]
        cols_ref[(0 * 3 + kw) * C:(0 * 3 + kw + 1) * C] = (
            jnp.concatenate([zrow, t[:, :HW - W]], axis=1))
        cols_ref[(1 * 3 + kw) * C:(1 * 3 + kw + 1) * C] = t
        cols_ref[(2 * 3 + kw) * C:(2 * 3 + kw + 1) * C] = (
            jnp.concatenate([t[:, W:], zrow], axis=1))


def _fold_scale_shift(st, count, gamma, beta):
    s = jnp.sum(st[:, :, 0], axis=0)
    ss = jnp.sum(st[:, :, 1], axis=0)
    mean = s / count
    var = ss / count - mean * mean
    scale = gamma * jax.lax.rsqrt(var + _EPS)
    shift = beta - mean * scale
    return scale.reshape(-1, 1), shift.reshape(-1, 1)


_CHUNK = 1024


def _dot_epilogue(w_ref, cols, o_ref, st_ref, i):
    C = o_ref.shape[1]
    HW = o_ref.shape[2]
    s = jnp.zeros((C, 1), jnp.float32)
    ss = jnp.zeros((C, 1), jnp.float32)
    for c0 in range(0, HW, _CHUNK):
        c1 = min(c0 + _CHUNK, HW)
        acc = jnp.dot(w_ref[...], cols[:, c0:c1],
                      preferred_element_type=jnp.float32)
        o_ref[i, :, c0:c1] = acc.astype(jnp.bfloat16)
        s = s + jnp.sum(acc, axis=1, keepdims=True)
        ss = ss + jnp.sum(acc * acc, axis=1, keepdims=True)
    st_ref[i, :, 0:1] = s
    st_ref[i, :, 1:2] = ss


def _conv1_kernel(W, x_ref, w_ref, o_ref, st_ref, cols_ref):
    for i in range(_IPB):
        xb = x_ref[i].astype(jnp.bfloat16)
        _build_cols(xb, W, cols_ref.at[i])
        _dot_epilogue(w_ref, cols_ref[i], o_ref, st_ref, i)


def _conv2_kernel(W, x_ref, sc_ref, sh_ref, w_ref, o_ref, st_ref, cols_ref):
    sc = sc_ref[...]
    sh = sh_ref[...]
    for i in range(_IPB):
        y = x_ref[i].astype(jnp.float32) * sc + sh
        yb = jnp.maximum(y, 0.0).astype(jnp.bfloat16)
        _build_cols(yb, W, cols_ref.at[i])
        _dot_epilogue(w_ref, cols_ref[i], o_ref, st_ref, i)


def _tail_kernel(x_ref, r_ref, sc_ref, sh_ref, o_ref):
    C = sc_ref.shape[0]
    x = x_ref[...].astype(jnp.float32)
    o_ref[...] = jnp.maximum(
        x * sc_ref[...].reshape(1, C, 1) + sh_ref[...].reshape(1, C, 1)
        + r_ref[...], 0.0)


def _conv1_bnstats(x_flat, w_t, W):
    N, C, HW = x_flat.shape
    KC = w_t.shape[1]
    return pl.pallas_call(
        functools.partial(_conv1_kernel, W),
        grid=(N // _IPB,),
        in_specs=[
            pl.BlockSpec((_IPB, C, HW), lambda n: (n, 0, 0)),
            pl.BlockSpec((C, KC), lambda n: (0, 0)),
        ],
        out_specs=(
            pl.BlockSpec((_IPB, C, HW), lambda n: (n, 0, 0)),
            pl.BlockSpec((_IPB, C, 2), lambda n: (n, 0, 0)),
        ),
        out_shape=(
            jax.ShapeDtypeStruct((N, C, HW), jnp.bfloat16),
            jax.ShapeDtypeStruct((N, C, 2), jnp.float32),
        ),
        scratch_shapes=[pltpu.VMEM((_IPB, KC, HW), jnp.bfloat16)],
        compiler_params=_cparams(("parallel",)),
    )(x_flat, w_t)


def _conv2_bnstats(c1, w_t, W, scale, shift):
    N, C, HW = c1.shape
    KC = w_t.shape[1]
    return pl.pallas_call(
        functools.partial(_conv2_kernel, W),
        grid=(N // _IPB,),
        in_specs=[
            pl.BlockSpec((_IPB, C, HW), lambda n: (n, 0, 0)),
            pl.BlockSpec((C, 1), lambda n: (0, 0)),
            pl.BlockSpec((C, 1), lambda n: (0, 0)),
            pl.BlockSpec((C, KC), lambda n: (0, 0)),
        ],
        out_specs=(
            pl.BlockSpec((_IPB, C, HW), lambda n: (n, 0, 0)),
            pl.BlockSpec((_IPB, C, 2), lambda n: (n, 0, 0)),
        ),
        out_shape=(
            jax.ShapeDtypeStruct((N, C, HW), jnp.bfloat16),
            jax.ShapeDtypeStruct((N, C, 2), jnp.float32),
        ),
        scratch_shapes=[pltpu.VMEM((_IPB, KC, HW), jnp.bfloat16)],
        compiler_params=_cparams(("parallel",)),
    )(c1, scale, shift, w_t)


def kernel(x_nchw, w1, w2, g1, b1, g2, b2):
    N, C, H, W = x_nchw.shape
    HW = H * W
    count = float(N * HW)

    w1t = w1.reshape(9 * C, C).T.astype(jnp.bfloat16)
    w2t = w2.reshape(9 * C, C).T.astype(jnp.bfloat16)
    x = x_nchw.reshape(N, C, HW)
    c1, st1 = _conv1_bnstats(x, w1t, W)
    sc1, sh1 = _fold_scale_shift(st1, count, g1, b1)
    c2, st2 = _conv2_bnstats(c1, w2t, W, sc1, sh1)
    sc2, sh2 = _fold_scale_shift(st2, count, g2, b2)

    out = pl.pallas_call(
        _tail_kernel,
        grid=(N // _IPB_TAIL,),
        in_specs=[
            pl.BlockSpec((_IPB_TAIL, C, HW), lambda n: (n, 0, 0)),
            pl.BlockSpec((_IPB_TAIL, C, HW), lambda n: (n, 0, 0)),
            pl.BlockSpec((C, 1), lambda n: (0, 0)),
            pl.BlockSpec((C, 1), lambda n: (0, 0)),
        ],
        out_specs=pl.BlockSpec((_IPB_TAIL, C, HW), lambda n: (n, 0, 0)),
        out_shape=jax.ShapeDtypeStruct((N, C, HW), jnp.float32),
        compiler_params=_cparams(("parallel",)),
    )(c2, x, sc2, sh2)
    return out.reshape(N, C, H, W)


# speedup vs baseline: 1.0252x; 1.0252x over previous
import functools

import jax
import jax.numpy as jnp
from jax.experimental import pallas as pl
from jax.experimental.pallas import tpu as pltpu

_EPS = 1e-5
_IPB = 2
_IPB_TAIL = 8
_VMEM_LIMIT = 64 * 1024 * 1024
_CP = getattr(pltpu, "CompilerParams", None) or getattr(
    pltpu, "TPUCompilerParams")


def _cparams(sem):
    return _CP(dimension_semantics=sem, vmem_limit_bytes=_VMEM_LIMIT)


def _build_cols(xb, W, cols_ref):
    C, HW = xb.shape
    col = jax.lax.broadcasted_iota(jnp.int32, (1, HW), 1) % W
    has_l = col != 0
    has_r = col != (W - 1)
    zrow = jnp.zeros((C, W), jnp.bfloat16)
    z1 = jnp.zeros((C, 1), jnp.bfloat16)
    zb = jnp.zeros((C, HW), jnp.bfloat16)
    taps = (
        jnp.where(has_l, jnp.concatenate([z1, xb[:, :HW - 1]], axis=1), zb),
        xb,
        jnp.where(has_r, jnp.concatenate([xb[:, 1:], z1], axis=1), zb),
    )
    for kw in range(3):
        t = taps[kw]
        cols_ref[(0 * 3 + kw) * C:(0 * 3 + kw + 1) * C] = (
            jnp.concatenate([zrow, t[:, :HW - W]], axis=1))
        cols_ref[(1 * 3 + kw) * C:(1 * 3 + kw + 1) * C] = t
        cols_ref[(2 * 3 + kw) * C:(2 * 3 + kw + 1) * C] = (
            jnp.concatenate([t[:, W:], zrow], axis=1))


def _fold_scale_shift(st, count, gamma, beta):
    s = jnp.sum(st[:, :, 0], axis=0)
    ss = jnp.sum(st[:, :, 1], axis=0)
    mean = s / count
    var = ss / count - mean * mean
    scale = gamma * jax.lax.rsqrt(var + _EPS)
    shift = beta - mean * scale
    return scale.reshape(-1, 1), shift.reshape(-1, 1)


def _conv1_kernel(W, x_ref, w_ref, o_ref, st_ref, cols_ref):
    for i in range(_IPB):
        xb = x_ref[i].astype(jnp.bfloat16)
        _build_cols(xb, W, cols_ref.at[i])
        acc = jnp.dot(w_ref[...], cols_ref[i],
                      preferred_element_type=jnp.float32)
        o_ref[i] = acc.astype(jnp.bfloat16)
        st_ref[i, :, 0:1] = jnp.sum(acc, axis=1, keepdims=True)
        st_ref[i, :, 1:2] = jnp.sum(acc * acc, axis=1, keepdims=True)


def _conv2_kernel(W, x_ref, sc_ref, sh_ref, w_ref, o_ref, st_ref, cols_ref):
    sc = sc_ref[...]
    sh = sh_ref[...]
    for i in range(_IPB):
        y = x_ref[i].astype(jnp.float32) * sc + sh
        yb = jnp.maximum(y, 0.0).astype(jnp.bfloat16)
        _build_cols(yb, W, cols_ref.at[i])
        acc = jnp.dot(w_ref[...], cols_ref[i],
                      preferred_element_type=jnp.float32)
        o_ref[i] = acc.astype(jnp.bfloat16)
        st_ref[i, :, 0:1] = jnp.sum(acc, axis=1, keepdims=True)
        st_ref[i, :, 1:2] = jnp.sum(acc * acc, axis=1, keepdims=True)


def _tail_kernel(x_ref, r_ref, sc_ref, sh_ref, o_ref):
    C = sc_ref.shape[0]
    x = x_ref[...].astype(jnp.float32)
    o_ref[...] = jnp.maximum(
        x * sc_ref[...].reshape(1, C, 1) + sh_ref[...].reshape(1, C, 1)
        + r_ref[...], 0.0)


def _conv1_bnstats(x_flat, w_t, W):
    N, C, HW = x_flat.shape
    KC = w_t.shape[1]
    return pl.pallas_call(
        functools.partial(_conv1_kernel, W),
        grid=(N // _IPB,),
        in_specs=[
            pl.BlockSpec((_IPB, C, HW), lambda n: (n, 0, 0)),
            pl.BlockSpec((C, KC), lambda n: (0, 0)),
        ],
        out_specs=(
            pl.BlockSpec((_IPB, C, HW), lambda n: (n, 0, 0)),
            pl.BlockSpec((_IPB, C, 2), lambda n: (n, 0, 0)),
        ),
        out_shape=(
            jax.ShapeDtypeStruct((N, C, HW), jnp.bfloat16),
            jax.ShapeDtypeStruct((N, C, 2), jnp.float32),
        ),
        scratch_shapes=[pltpu.VMEM((_IPB, KC, HW), jnp.bfloat16)],
        compiler_params=_cparams(("parallel",)),
    )(x_flat, w_t)


def _conv2_bnstats(c1, w_t, W, scale, shift):
    N, C, HW = c1.shape
    KC = w_t.shape[1]
    return pl.pallas_call(
        functools.partial(_conv2_kernel, W),
        grid=(N // _IPB,),
        in_specs=[
            pl.BlockSpec((_IPB, C, HW), lambda n: (n, 0, 0)),
            pl.BlockSpec((C, 1), lambda n: (0, 0)),
            pl.BlockSpec((C, 1), lambda n: (0, 0)),
            pl.BlockSpec((C, KC), lambda n: (0, 0)),
        ],
        out_specs=(
            pl.BlockSpec((_IPB, C, HW), lambda n: (n, 0, 0)),
            pl.BlockSpec((_IPB, C, 2), lambda n: (n, 0, 0)),
        ),
        out_shape=(
            jax.ShapeDtypeStruct((N, C, HW), jnp.bfloat16),
            jax.ShapeDtypeStruct((N, C, 2), jnp.float32),
        ),
        scratch_shapes=[pltpu.VMEM((_IPB, KC, HW), jnp.bfloat16)],
        compiler_params=_cparams(("parallel",)),
    )(c1, scale, shift, w_t)


def kernel(x_nchw, w1, w2, g1, b1, g2, b2):
    N, C, H, W = x_nchw.shape
    HW = H * W
    count = float(N * HW)

    w1t = w1.reshape(9 * C, C).T.astype(jnp.bfloat16)
    w2t = w2.reshape(9 * C, C).T.astype(jnp.bfloat16)
    x = x_nchw.reshape(N, C, HW)
    c1, st1 = _conv1_bnstats(x, w1t, W)
    sc1, sh1 = _fold_scale_shift(st1, count, g1, b1)
    c2, st2 = _conv2_bnstats(c1, w2t, W, sc1, sh1)
    sc2, sh2 = _fold_scale_shift(st2, count, g2, b2)

    out = pl.pallas_call(
        _tail_kernel,
        grid=(N // _IPB_TAIL,),
        in_specs=[
            pl.BlockSpec((_IPB_TAIL, C, HW), lambda n: (n, 0, 0)),
            pl.BlockSpec((_IPB_TAIL, C, HW), lambda n: (n, 0, 0)),
            pl.BlockSpec((C, 1), lambda n: (0, 0)),
            pl.BlockSpec((C, 1), lambda n: (0, 0)),
        ],
        out_specs=pl.BlockSpec((_IPB_TAIL, C, HW), lambda n: (n, 0, 0)),
        out_shape=jax.ShapeDtypeStruct((N, C, HW), jnp.float32),
        compiler_params=_cparams(("parallel",)),
    )(c2, x, sc2, sh2)
    return out.reshape(N, C, H, W)
